# baseline (speedup 1.0000x reference)
import functools

import jax
import jax.numpy as jnp
from jax import lax
from jax.experimental import pallas as pl
from jax.experimental.pallas import tpu as pltpu
from jax.experimental.pallas import tpu_sc as plsc

_NC = 2
_NS = 16
_NW = _NC * _NS


@functools.partial(jax.jit, static_argnums=(2,))
def _gather(vectors, x, rows_per_chunk):
    B, H = x.shape
    D = vectors.shape[1]
    rows_per_w = B // _NW
    n_chunks = rows_per_w // rows_per_chunk
    n_pairs = n_chunks // 2
    R = rows_per_chunk
    HP = (H + 7) // 8 * 8
    C = R * HP
    mesh = plsc.VectorSubcoreMesh(core_axis_name="c", subcore_axis_name="s")

    @functools.partial(
        pl.kernel,
        mesh=mesh,
        out_type=jax.ShapeDtypeStruct((B, H, D), jnp.float32),
        scratch_types=[
            pltpu.VMEM((C,), jnp.int32),
            pltpu.VMEM((C,), jnp.int32),
            pltpu.VMEM((C, D), jnp.float32),
            pltpu.VMEM((C, D), jnp.float32),
            pltpu.SemaphoreType.DMA,
            pltpu.SemaphoreType.DMA,
            pltpu.SemaphoreType.DMA,
            pltpu.SemaphoreType.DMA,
            pltpu.SemaphoreType.DMA,
            pltpu.SemaphoreType.DMA,
        ],
        compiler_params=pltpu.CompilerParams(use_tc_tiling_on_sc=False),
    )
    def k(table_hbm, idx_hbm, out_hbm, idx0, idx1, rows0, rows1,
          is0, is1, gs0, gs1, os0, os1):
        idxs, rows = (idx0, idx1), (rows0, rows1)
        isem, gsem, osem = (is0, is1), (gs0, gs1), (os0, os1)
        wid = lax.axis_index("s") * _NC + lax.axis_index("c")
        base = wid * rows_per_w

        def fire_idx(g, b):
            for r in range(R):
                pltpu.async_copy(
                    idx_hbm.at[base + g * R + r],
                    idxs[b].at[pl.ds(r * HP, H)], isem[b])

        def wait_idx(b):
            for r in range(R):
                pltpu.make_async_copy(
                    idx_hbm.at[base],
                    idxs[b].at[pl.ds(r * HP, H)], isem[b]).wait()

        def gather(b):
            pltpu.async_copy(table_hbm.at[idxs[b]], rows[b], gsem[b]).wait()

        def fire_out(g, b):
            for r in range(R):
                pltpu.async_copy(
                    rows[b].at[pl.ds(r * HP, H), :],
                    out_hbm.at[base + g * R + r], osem[b])

        def wait_out(b):
            for r in range(R):
                pltpu.make_async_copy(
                    rows[b].at[pl.ds(r * HP, H), :],
                    out_hbm.at[base], osem[b]).wait()

        zeros = jnp.zeros((16,), jnp.int32)
        for b in range(2):
            for i in range(C // 16):
                idxs[b][pl.ds(i * 16, 16)] = zeros

        fire_idx(0, 0)
        fire_idx(1, 1)
        for b in range(2):
            wait_idx(b)
            gather(b)
            fire_idx(b + 2, b)
            fire_out(b, b)

        def body(j, carry):
            for b in range(2):
                g = 2 * j + b

                wait_idx(b)
                wait_out(b)
                gather(b)

                @pl.when(g + 2 < n_chunks)
                def _():
                    fire_idx(g + 2, b)

                fire_out(g, b)
            return carry

        lax.fori_loop(1, n_pairs, body, 0)
        wait_out(0)
        wait_out(1)

    return k(vectors, x)


def kernel(x, vectors):
    return _gather(vectors, x, 32)


# speedup vs baseline: 1.8596x; 1.8596x over previous
import functools

import jax
import jax.numpy as jnp
from jax import lax
from jax.experimental import pallas as pl
from jax.experimental.pallas import tpu as pltpu
from jax.experimental.pallas import tpu_sc as plsc

_NC = 2
_NS = 16
_NW = _NC * _NS


@jax.jit
def _gather(vectors, x):
    B, H = x.shape
    D = vectors.shape[1]
    BPW = B // _NW
    BW = BPW // 2
    NBT = BW // 128
    HW = 2
    NH = H // HW
    NCHUNK = 2 * NH
    C = HW * BW
    mesh = plsc.VectorSubcoreMesh(core_axis_name="c", subcore_axis_name="s")

    @functools.partial(
        pl.kernel,
        mesh=mesh,
        out_type=jax.ShapeDtypeStruct((H, D // 8, B // 128, 8, 128),
                                      jnp.float32),
        scratch_types=[
            pltpu.VMEM((BPW, H), jnp.int32),
            pltpu.VMEM((C,), jnp.int32),
            pltpu.VMEM((C,), jnp.int32),
            pltpu.VMEM((C, D), jnp.float32),
            pltpu.VMEM((C, D), jnp.float32),
            pltpu.VMEM((HW, D // 8, NBT, 8, 128), jnp.float32),
            pltpu.VMEM((HW, D // 8, NBT, 8, 128), jnp.float32),
            pltpu.SemaphoreType.DMA,
            pltpu.SemaphoreType.DMA,
            pltpu.SemaphoreType.DMA,
            pltpu.SemaphoreType.DMA,
        ],
        compiler_params=pltpu.CompilerParams(
            use_tc_tiling_on_sc=False, needs_layout_passes=False),
    )
    def k(table_hbm, x_hbm, out_hbm, xv, il0, il1, rw0, rw1, tr0, tr1,
          gs0, gs1, os0, os1):
        ilist, rows, trans = (il0, il1), (rw0, rw1), (tr0, tr1)
        gsem, osem = (gs0, gs1), (os0, os1)
        wid = lax.axis_index("s") * _NC + lax.axis_index("c")
        b_base = wid * BPW
        bt_base = wid * (BPW // 128)
        lanes = lax.iota(jnp.int32, 16)


        def build_ilist(hwin, w, b):
            h0 = hwin * HW
            for hrel in range(HW):
                col = jnp.full((16,), 0, jnp.int32) + (h0 + hrel)
                for grp in range(BW // 16):
                    rvec = lanes + (w * BW + grp * 16)
                    v = plsc.load_gather(xv, [rvec, col])
                    ilist[b][pl.ds(hrel * BW + grp * 16, 16)] = v

        def fire_gather(b):
            pltpu.async_copy(table_hbm.at[ilist[b]], rows[b], gsem[b])

        def wait_gather(b):
            pltpu.make_async_copy(
                table_hbm.at[ilist[b]], rows[b], gsem[b]).wait()

        def transpose(b):
            def body(i, carry):
                dt = i >> 3
                dr = i & 7
                col = jnp.full((16,), 0, jnp.int32) + i
                for hrel in range(HW):
                    for btr in range(NBT):
                        for blg in range(8):
                            rvec = lanes + (hrel * BW + btr * 128 + blg * 16)
                            v = plsc.load_gather(rows[b], [rvec, col])
                            trans[b][hrel, dt, btr, dr,
                                     pl.ds(blg * 16, 16)] = v
                return carry

            lax.fori_loop(0, D, body, 0)

        def fire_out(hwin, w, b):
            pltpu.async_copy(
                trans[b],
                out_hbm.at[pl.ds(hwin * HW, HW), :,
                           pl.ds(bt_base + w * NBT, NBT), :, :],
                osem[b])

        def wait_out(b):
            pltpu.make_async_copy(
                trans[b],
                out_hbm.at[pl.ds(0, HW), :, pl.ds(bt_base, NBT), :, :],
                osem[b]).wait()

        pltpu.sync_copy(x_hbm.at[pl.ds(b_base, BPW)], xv)

        build_ilist(0, 0, 0)
        fire_gather(0)

        def pair(jj, carry):
            for w in range(2):
                b = w
                hwin = jj
                if w == 0:
                    build_ilist(jj, 1, 1)
                    fire_gather(1)
                else:
                    @pl.when(jj < NH - 1)
                    def _():
                        build_ilist(jj + 1, 0, 0)
                        fire_gather(0)

                wait_gather(b)

                @pl.when(jj > 0)
                def _():
                    wait_out(b)

                transpose(b)
                fire_out(hwin, w, b)
            return carry

        lax.fori_loop(0, NH, pair, 0)
        wait_out(0)
        wait_out(1)

    return k(vectors, x)


def kernel(x, vectors):
    B, H = x.shape
    D = vectors.shape[1]
    out_l = _gather(vectors, x)
    return out_l.transpose(2, 4, 0, 1, 3).reshape(B, H, D)
